# baseline (speedup 1.0000x reference)
import functools

import jax
import jax.numpy as jnp
from jax import lax
from jax.experimental import pallas as pl
from jax.experimental.pallas import tpu as pltpu
from jax.experimental.pallas import tpu_sc as plsc

_D = 48
_W = 128
_L = 16
_NCH = _W // _L
_B_IN = 32
_PAD = 128


def kernel(left, right):
    n, c, h, w = left.shape
    assert w == _W
    rows = n * 2 * c * h
    src = jnp.concatenate([left, right], axis=1).reshape(rows, _W)

    info = plsc.get_sparse_core_info()
    num_cores = info.num_cores
    num_workers = num_cores * info.num_subcores
    rpw = rows // num_workers
    ch_half = c * h

    mesh = plsc.VectorSubcoreMesh(core_axis_name="c", subcore_axis_name="s")

    @functools.partial(
        pl.kernel,
        mesh=mesh,
        compiler_params=pltpu.CompilerParams(needs_layout_passes=False),
        out_type=jax.ShapeDtypeStruct((rows, _D, _W), jnp.float32),
        scratch_types=[
            pltpu.VMEM((_B_IN, _PAD + _W), jnp.float32),
            pltpu.VMEM((_D, _W), jnp.float32),
            pltpu.VMEM((_D, _W), jnp.float32),
            pltpu.SemaphoreType.DMA,
            pltpu.SemaphoreType.DMA,
        ],
    )
    def sc_body(src_hbm, out_hbm, src_big, out0, out1, sem0, sem1):
        wid = lax.axis_index("s") * num_cores + lax.axis_index("c")
        base = wid * rpw
        half = (base // ch_half) % 2
        iota = lax.broadcasted_iota(jnp.int32, (_L,), 0)
        zeros = jnp.zeros((_L,), jnp.float32)

        for rr in range(_B_IN):
            for mm in range(_PAD // _L):
                src_big[rr, pl.ds(_L * mm, _L)] = zeros

        def left_row(br, outb):
            lv = [src_big[br, pl.ds(_PAD + _L * m, _L)] for m in range(_NCH)]
            for d in range(_D):
                mp = d // _L
                for m in range(_NCH):
                    if m < mp:
                        outb[d, pl.ds(_L * m, _L)] = zeros
                    elif m == mp and d % _L != 0:
                        outb[d, pl.ds(_L * m, _L)] = jnp.where(
                            iota >= d - _L * m, lv[m], 0.0)
                    else:
                        outb[d, pl.ds(_L * m, _L)] = lv[m]

        def right_row(br, outb):
            row_splat = jnp.full((_L,), br, jnp.int32)
            for d in range(_D):
                k = d % _L
                mp = d // _L
                for m in range(_NCH):
                    if m < mp:
                        outb[d, pl.ds(_L * m, _L)] = zeros
                    elif m == mp and k != 0:
                        cols = _PAD + jnp.maximum(iota - k, 0)
                        vals = plsc.load_gather(src_big, [row_splat, cols])
                        outb[d, pl.ds(_L * m, _L)] = jnp.where(
                            iota >= k, vals, 0.0)
                    else:
                        outb[d, pl.ds(_L * m, _L)] = src_big[
                            br, pl.ds(_PAD + _L * m - d, _L)]

        def run(build_row):
            def pair(i2, carry):
                i = 2 * i2

                @pl.when(lax.rem(i, _B_IN) == 0)
                def _stage():
                    off = pl.multiple_of(base + i, _B_IN)
                    pltpu.sync_copy(src_hbm.at[pl.ds(off, _B_IN)],
                                    src_big.at[:, pl.ds(_PAD, _W)])

                for b, (outb, semb) in enumerate(((out0, sem0),
                                                  (out1, sem1))):
                    r = base + i + b
                    br = lax.rem(i, _B_IN) + b

                    @pl.when(i2 > 0)
                    def _drain():
                        pltpu.make_async_copy(
                            outb, out_hbm.at[r], semb).wait()

                    build_row(br, outb)
                    pltpu.make_async_copy(outb, out_hbm.at[r], semb).start()
                return carry

            lax.fori_loop(0, rpw // 2, pair, 0)
            pltpu.make_async_copy(out0, out_hbm.at[base], sem0).wait()
            pltpu.make_async_copy(out1, out_hbm.at[base], sem1).wait()

        @pl.when(half == 0)
        def _left():
            run(left_row)

        @pl.when(half == 1)
        def _right():
            run(right_row)

    out = sc_body(src)
    return jnp.swapaxes(out.reshape(n, 2 * c, h, _D, _W), -1, -2)


# speedup vs baseline: 2.8725x; 2.8725x over previous
import functools

import jax
import jax.numpy as jnp
from jax import lax
from jax.experimental import pallas as pl
from jax.experimental.pallas import tpu as pltpu
from jax.experimental.pallas import tpu_sc as plsc

_D = 48
_W = 128
_L = 16
_NCH = _W // _L
_B_IN = 32
_PAD = 128
_R_OUT = 4


def kernel(left, right):
    n, c, h, w = left.shape
    assert w == _W
    rows = n * 2 * c * h
    src = jnp.concatenate([left, right], axis=1).reshape(rows, _W)

    info = plsc.get_sparse_core_info()
    num_cores = info.num_cores
    num_workers = num_cores * info.num_subcores
    rpw = rows // num_workers
    ch_half = c * h

    mesh = plsc.VectorSubcoreMesh(core_axis_name="c", subcore_axis_name="s")

    @functools.partial(
        pl.kernel,
        mesh=mesh,
        compiler_params=pltpu.CompilerParams(needs_layout_passes=False),
        out_type=jax.ShapeDtypeStruct((rows, _D, _W), jnp.float32),
        scratch_types=[
            pltpu.VMEM((_B_IN, _PAD + _W), jnp.float32),
            pltpu.VMEM((_R_OUT, _D, _W), jnp.float32),
            pltpu.VMEM((_R_OUT, _D, _W), jnp.float32),
            pltpu.SemaphoreType.DMA,
            pltpu.SemaphoreType.DMA,
        ],
    )
    def sc_body(src_hbm, out_hbm, src_big, out0, out1, sem0, sem1):
        wid = lax.axis_index("s") * num_cores + lax.axis_index("c")
        base = wid * rpw
        half = (base // ch_half) % 2
        iota = lax.broadcasted_iota(jnp.int32, (_L,), 0)
        zeros = jnp.zeros((_L,), jnp.float32)

        for rr in range(_B_IN):
            for mm in range(_PAD // _L):
                src_big[rr, pl.ds(_L * mm, _L)] = zeros

        for outb in (out0, out1):
            def _zinit(rr, carry):
                for d in range(_D):
                    for m in range(d // _L):
                        outb[rr, d, pl.ds(_L * m, _L)] = zeros
                return carry
            lax.fori_loop(0, _R_OUT, _zinit, 0)

        def left_row(br, rr, outb):
            lv = [src_big[br, pl.ds(_PAD + _L * m, _L)] for m in range(_NCH)]
            for d in range(_D):
                k = d % _L
                mp = d // _L
                for m in range(mp, _NCH):
                    if m == mp and k != 0:
                        outb[rr, d, pl.ds(_L * m, _L)] = jnp.where(
                            iota >= k, lv[m], 0.0)
                    else:
                        outb[rr, d, pl.ds(_L * m, _L)] = lv[m]

        def right_row(br, rr, outb):
            row_splat = jnp.full((_L,), br, jnp.int32)
            for k in range(_L):
                vs = []
                for m in range(_NCH):
                    if m == 0 and k != 0:
                        cols = _PAD + jnp.maximum(iota - k, 0)
                        g = plsc.load_gather(src_big, [row_splat, cols])
                        vs.append(jnp.where(iota >= k, g, 0.0))
                    else:
                        vs.append(src_big[
                            br, pl.ds(_PAD + _L * m - k, _L)])
                for j in range(3):
                    d = k + _L * j
                    for m in range(j, _NCH):
                        outb[rr, d, pl.ds(_L * m, _L)] = vs[m - j]

        def run(build_row):
            blocks_per_stage = _B_IN // (2 * _R_OUT)

            def block(it, carry):
                i = 2 * _R_OUT * it

                @pl.when(lax.rem(it, blocks_per_stage) == 0)
                def _stage():
                    off = pl.multiple_of(base + i, _B_IN)
                    pltpu.sync_copy(src_hbm.at[pl.ds(off, _B_IN)],
                                    src_big.at[:, pl.ds(_PAD, _W)])

                for s, (outb, semb) in enumerate(((out0, sem0),
                                                  (out1, sem1))):
                    r0 = base + i + _R_OUT * s

                    @pl.when(it > 0)
                    def _drain():
                        pltpu.make_async_copy(
                            outb, out_hbm.at[pl.ds(r0, _R_OUT)], semb).wait()

                    def one(rr, c):
                        br = lax.rem(i, _B_IN) + _R_OUT * s + rr
                        build_row(br, rr, outb)
                        return c

                    lax.fori_loop(0, _R_OUT, one, 0)
                    pltpu.make_async_copy(
                        outb, out_hbm.at[pl.ds(r0, _R_OUT)], semb).start()
                return carry

            lax.fori_loop(0, rpw // (2 * _R_OUT), block, 0)
            pltpu.make_async_copy(
                out0, out_hbm.at[pl.ds(base, _R_OUT)], sem0).wait()
            pltpu.make_async_copy(
                out1, out_hbm.at[pl.ds(base, _R_OUT)], sem1).wait()

        @pl.when(half == 0)
        def _left():
            run(left_row)

        @pl.when(half == 1)
        def _right():
            run(right_row)

    out = sc_body(src)
    return jnp.swapaxes(out.reshape(n, 2 * c, h, _D, _W), -1, -2)


# speedup vs baseline: 3.4266x; 1.1929x over previous
import functools

import jax
import jax.numpy as jnp
from jax import lax
from jax.experimental import pallas as pl
from jax.experimental.pallas import tpu as pltpu
from jax.experimental.pallas import tpu_sc as plsc

_D = 48
_W = 128
_L = 16
_NCH = _W // _L
_B_IN = 32
_PAD = 128
_R_OUT = 8


def kernel(left, right):
    n, c, h, w = left.shape
    assert w == _W
    rows = n * 2 * c * h
    src = jnp.concatenate([left, right], axis=1).reshape(rows, _W)

    info = plsc.get_sparse_core_info()
    num_cores = info.num_cores
    num_workers = num_cores * info.num_subcores
    rpw = rows // num_workers
    ch_half = c * h

    mesh = plsc.VectorSubcoreMesh(core_axis_name="c", subcore_axis_name="s")

    @functools.partial(
        pl.kernel,
        mesh=mesh,
        compiler_params=pltpu.CompilerParams(needs_layout_passes=False),
        out_type=jax.ShapeDtypeStruct((rows, _D, _W), jnp.float32),
        scratch_types=[
            pltpu.VMEM((_B_IN, _PAD + _W), jnp.float32),
            pltpu.VMEM((_R_OUT, _D, _W), jnp.float32),
            pltpu.VMEM((_R_OUT, _D, _W), jnp.float32),
            pltpu.SemaphoreType.DMA,
            pltpu.SemaphoreType.DMA,
        ],
    )
    def sc_body(src_hbm, out_hbm, src_big, out0, out1, sem0, sem1):
        wid = lax.axis_index("s") * num_cores + lax.axis_index("c")
        base = wid * rpw
        half = (base // ch_half) % 2
        iota = lax.broadcasted_iota(jnp.int32, (_L,), 0)
        zeros = jnp.zeros((_L,), jnp.float32)

        for rr in range(_B_IN):
            for mm in range(_PAD // _L):
                src_big[rr, pl.ds(_L * mm, _L)] = zeros

        for outb in (out0, out1):
            def _zinit(rr, carry):
                for d in range(_D):
                    for m in range(d // _L):
                        outb[rr, d, pl.ds(_L * m, _L)] = zeros
                return carry
            lax.fori_loop(0, _R_OUT, _zinit, 0)

        def left_row(br, rr, outb):
            lv = [src_big[br, pl.ds(_PAD + _L * m, _L)] for m in range(_NCH)]
            for d in range(_D):
                k = d % _L
                mp = d // _L
                for m in range(mp, _NCH):
                    if m == mp and k != 0:
                        outb[rr, d, pl.ds(_L * m, _L)] = jnp.where(
                            iota >= k, lv[m], 0.0)
                    else:
                        outb[rr, d, pl.ds(_L * m, _L)] = lv[m]

        def right_row(br, rr, outb):
            row_splat = jnp.full((_L,), br, jnp.int32)
            for k in range(_L):
                vs = []
                for m in range(_NCH):
                    if m == 0 and k != 0:
                        cols = _PAD + jnp.maximum(iota - k, 0)
                        g = plsc.load_gather(src_big, [row_splat, cols])
                        vs.append(jnp.where(iota >= k, g, 0.0))
                    else:
                        vs.append(src_big[
                            br, pl.ds(_PAD + _L * m - k, _L)])
                for j in range(3):
                    d = k + _L * j
                    for m in range(j, _NCH):
                        outb[rr, d, pl.ds(_L * m, _L)] = vs[m - j]

        def run(build_row):
            blocks_per_stage = _B_IN // (2 * _R_OUT)

            def block(it, carry):
                i = 2 * _R_OUT * it

                @pl.when(lax.rem(it, blocks_per_stage) == 0)
                def _stage():
                    off = pl.multiple_of(base + i, _B_IN)
                    pltpu.sync_copy(src_hbm.at[pl.ds(off, _B_IN)],
                                    src_big.at[:, pl.ds(_PAD, _W)])

                for s, (outb, semb) in enumerate(((out0, sem0),
                                                  (out1, sem1))):
                    r0 = base + i + _R_OUT * s

                    @pl.when(it > 0)
                    def _drain():
                        pltpu.make_async_copy(
                            outb, out_hbm.at[pl.ds(r0, _R_OUT)], semb).wait()

                    def one(rr, c):
                        br = lax.rem(i, _B_IN) + _R_OUT * s + rr
                        build_row(br, rr, outb)
                        return c

                    lax.fori_loop(0, _R_OUT, one, 0)
                    pltpu.make_async_copy(
                        outb, out_hbm.at[pl.ds(r0, _R_OUT)], semb).start()
                return carry

            lax.fori_loop(0, rpw // (2 * _R_OUT), block, 0)
            pltpu.make_async_copy(
                out0, out_hbm.at[pl.ds(base, _R_OUT)], sem0).wait()
            pltpu.make_async_copy(
                out1, out_hbm.at[pl.ds(base, _R_OUT)], sem1).wait()

        @pl.when(half == 0)
        def _left():
            run(left_row)

        @pl.when(half == 1)
        def _right():
            run(right_row)

    out = sc_body(src)
    return jnp.swapaxes(out.reshape(n, 2 * c, h, _D, _W), -1, -2)
